# Initial kernel scaffold; baseline (speedup 1.0000x reference)
import jax
import jax.numpy as jnp
from jax.experimental import pallas as pl


def kernel(indices, table):
    raise NotImplementedError("write your pallas kernel here")


# speedup vs baseline: 1.6836x; 1.6836x over previous
import functools

import jax
import jax.numpy as jnp
from jax import lax
from jax.experimental import pallas as pl
from jax.experimental.pallas import tpu as pltpu
from jax.experimental.pallas import tpu_sc as plsc

_NC = 2
_NS = 16
_NW = _NC * _NS
_G = 128


@functools.lru_cache(maxsize=None)
def _build(B, V, D):
    n_g = B // (_NW * _G)
    b_per_w = n_g * _G
    mesh = plsc.VectorSubcoreMesh(core_axis_name="c", subcore_axis_name="s")

    @functools.partial(
        pl.kernel,
        out_type=jax.ShapeDtypeStruct((B, D), jnp.float32),
        mesh=mesh,
        scratch_types=[
            pltpu.VMEM((n_g, _G), jnp.int32),
            pltpu.VMEM((_G, D), jnp.float32),
            pltpu.SemaphoreType.DMA,
        ],
        compiler_params=pltpu.CompilerParams(use_tc_tiling_on_sc=False),
    )
    def k(idx_hbm, table_hbm, out_hbm, idx_v, rows_v, sem):
        wid = lax.axis_index("s") * _NC + lax.axis_index("c")
        base = wid * b_per_w
        pltpu.sync_copy(idx_hbm.at[wid], idx_v)

        def step(g, carry):
            pltpu.async_copy(table_hbm.at[idx_v.at[g]], rows_v, sem).wait()
            pltpu.sync_copy(rows_v, out_hbm.at[pl.ds(base + g * _G, _G)])
            return carry

        lax.fori_loop(0, n_g, step, 0)

    return k


def kernel(indices, table):
    B = indices.shape[0] * indices.shape[1]
    V, D = table.shape
    idx = indices.reshape(_NW, B // (_NW * _G), _G).astype(jnp.int32)
    out = _build(B, V, D)(idx, table)
    return out.reshape(indices.shape[0], indices.shape[1], D)


# speedup vs baseline: 1.8757x; 1.1141x over previous
import functools

import jax
import jax.numpy as jnp
from jax import lax
from jax.experimental import pallas as pl
from jax.experimental.pallas import tpu as pltpu
from jax.experimental.pallas import tpu_sc as plsc

_NC = 2
_NS = 16
_NW = _NC * _NS
_G = 128


@functools.lru_cache(maxsize=None)
def _build(B, V, D):
    n_g = B // (_NW * _G)
    b_per_w = n_g * _G
    mesh = plsc.VectorSubcoreMesh(core_axis_name="c", subcore_axis_name="s")

    nbuf = 8
    n_blk = n_g // nbuf

    @functools.partial(
        pl.kernel,
        out_type=jax.ShapeDtypeStruct((B, D), jnp.float32),
        mesh=mesh,
        scratch_types=[
            pltpu.VMEM((n_g, _G), jnp.int32),
            pltpu.VMEM((nbuf, _G, D), jnp.float32),
            pltpu.SemaphoreType.DMA,
            pltpu.SemaphoreType.DMA,
        ],
        compiler_params=pltpu.CompilerParams(use_tc_tiling_on_sc=False),
    )
    def k(idx_hbm, table_hbm, out_hbm, idx_v, rows_v, sem_in, sem_out):
        wid = lax.axis_index("s") * _NC + lax.axis_index("c")
        base = wid * b_per_w
        pltpu.sync_copy(idx_hbm.at[wid], idx_v)

        for b in range(nbuf):
            pltpu.async_copy(table_hbm.at[idx_v.at[b]], rows_v.at[b], sem_in)

        def blk(j, carry):
            for b in range(nbuf):
                g = j * nbuf + b
                pltpu.make_async_copy(
                    table_hbm.at[idx_v.at[b]], rows_v.at[b], sem_in).wait()
                pltpu.async_copy(
                    rows_v.at[b], out_hbm.at[pl.ds(base + g * _G, _G)], sem_out)
            for b in range(nbuf):
                pltpu.make_async_copy(
                    rows_v.at[b], out_hbm.at[pl.ds(base, _G)], sem_out).wait()

                @pl.when(j + 1 < n_blk)
                def _():
                    pltpu.async_copy(
                        table_hbm.at[idx_v.at[(j + 1) * nbuf + b]],
                        rows_v.at[b], sem_in)
            return carry

        lax.fori_loop(0, n_blk, blk, 0)

    return k


def kernel(indices, table):
    B = indices.shape[0] * indices.shape[1]
    V, D = table.shape
    idx = indices.reshape(_NW, B // (_NW * _G), _G).astype(jnp.int32)
    out = _build(B, V, D)(idx, table)
    return out.reshape(indices.shape[0], indices.shape[1], D)


# speedup vs baseline: 1.8816x; 1.0032x over previous
import functools

import jax
import jax.numpy as jnp
from jax import lax
from jax.experimental import pallas as pl
from jax.experimental.pallas import tpu as pltpu
from jax.experimental.pallas import tpu_sc as plsc

_NC = 2
_NS = 16
_NW = _NC * _NS
_G = 128
_BB = 512
_NBUF = 3


@functools.lru_cache(maxsize=None)
def _build(BATCH, HIST, V, D):
    n_g = _BB // _G
    n_bblk = BATCH // _BB
    n_units = n_bblk * HIST
    u_per_w = n_units // _NW
    mesh = plsc.VectorSubcoreMesh(core_axis_name="c", subcore_axis_name="s")

    @functools.partial(
        pl.kernel,
        out_type=jax.ShapeDtypeStruct((BATCH, HIST, D), jnp.float32),
        mesh=mesh,
        scratch_types=[
            pltpu.VMEM((_NBUF, n_g, _G), jnp.int32),
            pltpu.VMEM((_NBUF, _BB, D), jnp.float32),
            pltpu.SemaphoreType.DMA((_NBUF,)),
            pltpu.SemaphoreType.DMA((_NBUF,)),
            pltpu.SemaphoreType.DMA((_NBUF,)),
        ],
        compiler_params=pltpu.CompilerParams(use_tc_tiling_on_sc=False),
    )
    def k(idx_hbm, table_hbm, out_hbm, idx_v, rows_v, sem_idx, sem_in, sem_out):
        wid = lax.axis_index("s") * _NC + lax.axis_index("c")
        u0 = wid * u_per_w

        def start_idx(u, b):
            h, bb = u // n_bblk, lax.rem(u, n_bblk)
            for g in range(n_g):
                pltpu.async_copy(
                    idx_hbm.at[h, pl.ds(bb * _BB + g * _G, _G)],
                    idx_v.at[b, g], sem_idx.at[b])

        def start_gathers(u, b):
            for g in range(n_g):
                pltpu.make_async_copy(
                    idx_hbm.at[0, pl.ds(0, _G)],
                    idx_v.at[b, g], sem_idx.at[b]).wait()
            for g in range(n_g):
                pltpu.async_copy(
                    table_hbm.at[idx_v.at[b, g]],
                    rows_v.at[b, pl.ds(g * _G, _G)], sem_in.at[b])

        def wait_gathers(b):
            for g in range(n_g):
                pltpu.make_async_copy(
                    table_hbm.at[idx_v.at[b, g]],
                    rows_v.at[b, pl.ds(g * _G, _G)], sem_in.at[b]).wait()

        def start_out(u, b):
            h, bb = u // n_bblk, lax.rem(u, n_bblk)
            pltpu.async_copy(
                rows_v.at[b], out_hbm.at[pl.ds(bb * _BB, _BB), h], sem_out.at[b])

        def wait_out(b):
            pltpu.make_async_copy(
                rows_v.at[b], out_hbm.at[pl.ds(0, _BB), 0], sem_out.at[b]).wait()

        for b in range(_NBUF):
            start_idx(u0 + b, b)
        for b in range(_NBUF):
            start_gathers(u0 + b, b)

        def step(j, carry):
            b = lax.rem(j, _NBUF)
            wait_gathers(b)

            @pl.when(j + _NBUF < u_per_w)
            def _():
                start_idx(u0 + j + _NBUF, b)

            start_out(u0 + j, b)

            @pl.when(j + _NBUF < u_per_w)
            def _():
                wait_out(b)
                start_gathers(u0 + j + _NBUF, b)

            return carry

        lax.fori_loop(0, u_per_w, step, 0)
        for b in range(_NBUF):
            wait_out(b)

    return k


def kernel(indices, table):
    BATCH, HIST = indices.shape
    V, D = table.shape
    idx_t = jnp.transpose(indices).astype(jnp.int32)
    return _build(BATCH, HIST, V, D)(idx_t, table)
